# baseline (speedup 1.0000x reference)
import functools

import jax
import jax.numpy as jnp
from jax import lax
from jax.experimental import pallas as pl
from jax.experimental.pallas import tpu as pltpu
from jax.experimental.pallas import tpu_sc as plsc

NE = 8192
ED = 256
BM = 256
GRID = NE // BM
N_TOK = 8 * 1024
CHUNK = 2736


def _bf16(v):
    return v.astype(jnp.bfloat16).astype(jnp.float32)


def _dist_kernel(x_ref, w_ref, idx_ref, loss_ref, wsq_ref, acc_ref):
    i = pl.program_id(0)
    xt = x_ref[...]
    w = w_ref[...]

    @pl.when(i == 0)
    def _():
        wsq_ref[...] = jnp.sum(w * w, axis=1)
        acc_ref[0] = 0.0

    a = jnp.sum(xt * xt, axis=1)
    mm = lax.dot_general(xt, w, (((1,), (1,)), ((), ())),
                         preferred_element_type=jnp.float32)
    d = (a[:, None] + wsq_ref[...][None, :]) - 2.0 * mm
    jj = lax.broadcasted_iota(jnp.int32, d.shape, 1)
    inf = jnp.float32(jnp.inf)
    big = jnp.int32(NE)

    def chunk_argmin(lo, hi):
        mask = (jj >= lo) & (jj < hi)
        dc = jnp.where(mask, d, inf)
        m = jnp.min(dc, axis=1)
        ii = jnp.min(jnp.where((dc == m[:, None]) & mask, jj, big), axis=1)
        return m, ii

    m0, i0 = chunk_argmin(0, CHUNK)
    m1, i1 = chunk_argmin(CHUNK, 2 * CHUNK)
    m2, i2 = chunk_argmin(2 * CHUNK, NE)

    b0 = _bf16(m0)
    keep1 = (b0 < m1) | ((b0 == m1) & (i0 < i1))
    v1 = jnp.where(keep1, m0, m1)
    s1 = jnp.where(keep1, i0, i1)
    b1 = _bf16(jnp.where(b0 < m1, b0, m1))
    keep2 = (b1 < m2) | ((b1 == m2) & (s1 < i2))
    v2 = jnp.where(keep2, v1, m2)
    s2 = jnp.where(keep2, s1, i2)

    idx_ref[...] = s2
    acc_ref[0] += jnp.sum(v2)

    @pl.when(i == GRID - 1)
    def _():
        loss_ref[0] = 1.25 * acc_ref[0] / float(N_TOK * ED)


def _argmin_and_loss(x_flat, W, interpret=False):
    return pl.pallas_call(
        _dist_kernel,
        grid=(GRID,),
        in_specs=[
            pl.BlockSpec((BM, ED), lambda i: (i, 0)),
            pl.BlockSpec((NE, ED), lambda i: (0, 0)),
        ],
        out_specs=[
            pl.BlockSpec((BM,), lambda i: (i,)),
            pl.BlockSpec(memory_space=pltpu.SMEM),
        ],
        out_shape=[
            jax.ShapeDtypeStruct((N_TOK,), jnp.int32),
            jax.ShapeDtypeStruct((1,), jnp.float32),
        ],
        scratch_shapes=[
            pltpu.VMEM((NE,), jnp.float32),
            pltpu.SMEM((1,), jnp.float32),
        ],
        interpret=interpret,
    )(x_flat, W)


NW = 32
BPW = N_TOK // NW


def _make_sc_gather():
    mesh = plsc.VectorSubcoreMesh(core_axis_name="c", subcore_axis_name="s")

    @functools.partial(
        pl.kernel, mesh=mesh,
        out_type=jax.ShapeDtypeStruct((N_TOK, ED), jnp.float32),
        scratch_types=[
            pltpu.VMEM((2, 128), jnp.int32),
            pltpu.VMEM((BPW, ED), jnp.float32),
            pltpu.SemaphoreType.DMA,
        ],
    )
    def gather_k(table_hbm, idx_hbm, out_hbm, idx_v, rows_v, sem):
        wid = lax.axis_index("s") * 2 + lax.axis_index("c")
        pltpu.sync_copy(idx_hbm.at[pl.ds(wid * 2, 2)], idx_v)
        for c in range(2):
            pltpu.async_copy(table_hbm.at[idx_v.at[c]],
                             rows_v.at[pl.ds(c * 128, 128)], sem).wait()
        pltpu.sync_copy(rows_v, out_hbm.at[pl.ds(wid * BPW, BPW)])

    return gather_k


def kernel(x, W):
    x_flat = x.reshape(-1, ED)
    idx, loss = _argmin_and_loss(x_flat, W)
    q = _make_sc_gather()(W, idx.reshape(NW * 2, 128))
    return q.reshape(x.shape), loss[0]
